# Initial kernel scaffold; baseline (speedup 1.0000x reference)
import jax
import jax.numpy as jnp
from jax.experimental import pallas as pl


def kernel(x1_maps, x2_maps, x1_glob, x2_glob, x1_locations, x2_locations):
    raise NotImplementedError("write your pallas kernel here")


# speedup vs baseline: 2.2301x; 2.2301x over previous
import functools

import jax
import jax.numpy as jnp
from jax.experimental import pallas as pl

NUM_MATCHES = 20
ALPHA = 0.75
INV_COEFF = 25.0
VAR_COEFF = 15.0
COV_COEFF = 1.0
GAMMA = 1.0

B, P, D_LOC, D_GLOB = 32, 576, 384, 2048
BIG_I32 = 1 << 30


def _dist_matrix(a, b):
    a2 = jnp.sum(a * a, axis=1, keepdims=True)
    b2 = jnp.transpose(jnp.sum(b * b, axis=1, keepdims=True))
    ab = jax.lax.dot_general(
        a.astype(jnp.bfloat16), b.astype(jnp.bfloat16),
        (((1,), (1,)), ((), ())),
        preferred_element_type=jnp.float32)
    d2 = a2 + b2 - 2.0 * ab
    d2 = jnp.maximum(d2, 0.0)
    safe = jnp.where(d2 > 0, d2, 1.0)
    return jnp.where(d2 > 0, jnp.sqrt(safe), 0.0)


def _min_argmin(d, axis):
    iota = jax.lax.broadcasted_iota(jnp.int32, d.shape, axis)
    m = jnp.min(d, axis=axis, keepdims=True)
    arg = jnp.min(jnp.where(d == m, iota, BIG_I32), axis=axis)
    return jnp.min(d, axis=axis), arg


def _knn_body(x1m_ref, x2m_ref, x1l_ref, x2l_ref,
              frmin_ref, frarg_ref, fcmin_ref, fcarg_ref,
              lrmin_ref, lrarg_ref, lcmin_ref, lcarg_ref):
    fd = _dist_matrix(x1m_ref[0], x2m_ref[0])
    frmin_ref[0, 0, :], frarg_ref[0, 0, :] = _min_argmin(fd, 1)
    fcmin_ref[0, 0, :], fcarg_ref[0, 0, :] = _min_argmin(fd, 0)
    ld = _dist_matrix(x1l_ref[0], x2l_ref[0])
    lrmin_ref[0, 0, :], lrarg_ref[0, 0, :] = _min_argmin(ld, 1)
    lcmin_ref[0, 0, :], lcarg_ref[0, 0, :] = _min_argmin(ld, 0)


def _match_terms(vals, args, ch_in, ch_cand):
    iota = jax.lax.broadcasted_iota(jnp.int32, (B, P), 1)
    inv_acc = jnp.float32(0.0)
    vin_acc = jnp.float32(0.0)
    vcand_acc = jnp.float32(0.0)
    v = vals
    for _ in range(NUM_MATCHES):
        m = jnp.min(v, axis=1, keepdims=True)
        first = jnp.min(jnp.where(v == m, iota, BIG_I32),
                        axis=1, keepdims=True)
        onehot = iota == first
        fi = jnp.sum(jnp.where(onehot, ch_in, 0.0), axis=1, keepdims=True)
        cand = jnp.sum(jnp.where(onehot, args, 0), axis=1, keepdims=True)
        onehot2 = iota == cand
        fc = jnp.sum(jnp.where(onehot2, ch_cand, 0.0), axis=1, keepdims=True)
        inv_acc += jnp.sum((fi - fc) ** 2)
        mu_i = jnp.sum(fi) / B
        vin_acc += jnp.maximum(
            GAMMA - jnp.sqrt(jnp.sum((fi - mu_i) ** 2) / (B - 1)), 0.0)
        mu_c = jnp.sum(fc) / B
        vcand_acc += jnp.maximum(
            GAMMA - jnp.sqrt(jnp.sum((fc - mu_c) ** 2) / (B - 1)), 0.0)
        v = jnp.where(onehot, jnp.inf, v)
    inv = INV_COEFF * inv_acc / (B * NUM_MATCHES)
    var = VAR_COEFF * (vin_acc + vcand_acc) / (2.0 * NUM_MATCHES)
    return inv + var


def _global_half(x):
    xc = x - jnp.sum(x, axis=0, keepdims=True) / B
    xcc = xc - jnp.sum(xc, axis=0, keepdims=True) / B
    std = jnp.sqrt(jnp.sum(xcc * xcc, axis=0, keepdims=True) / (B - 1))
    var = jnp.sum(jnp.maximum(GAMMA - std, 0.0)) / D_GLOB
    xcb = xc.astype(jnp.bfloat16)
    xcb32 = xcb.astype(jnp.float32)
    colss = jnp.sum(xcb32 * xcb32, axis=0, keepdims=True)
    g = jax.lax.dot_general(
        xcb, xcb, (((1,), (1,)), ((), ())),
        preferred_element_type=jnp.float32)
    s_all = jnp.sum(g * g)
    s_diag = jnp.sum(colss * colss)
    cov = (s_all - s_diag) / ((B - 1.0) * (B - 1.0)) / D_GLOB
    return var, cov


def _loss_body(frmin_ref, frarg_ref, fcmin_ref, fcarg_ref,
               lrmin_ref, lrarg_ref, lcmin_ref, lcarg_ref,
               ch1_ref, ch2_ref, g1_ref, g2_ref,
               loss_ref, gl_ref, loc_ref, feat_ref):
    ch1 = ch1_ref[...]
    ch2 = ch2_ref[...]
    f12 = _match_terms(frmin_ref[...], frarg_ref[...], ch1, ch2)
    f21 = _match_terms(fcmin_ref[...], fcarg_ref[...], ch2, ch1)
    l12 = _match_terms(lrmin_ref[...], lrarg_ref[...], ch1, ch2)
    l21 = _match_terms(lcmin_ref[...], lcarg_ref[...], ch2, ch1)
    feat = (f12 + f21) / 2.0
    loc = (l12 + l21) / 2.0

    g1 = g1_ref[...]
    g2 = g2_ref[...]
    inv_g = INV_COEFF * jnp.sum((g1 - g2) ** 2) / (B * D_GLOB)
    var1, cov1 = _global_half(g1)
    var2, cov2 = _global_half(g2)
    global_loss = (inv_g + VAR_COEFF * (var1 + var2) / 2.0
                   + COV_COEFF * (cov1 + cov2) / 2.0)

    loss = ALPHA * global_loss + (1.0 - ALPHA) * (feat + loc) / 2.0
    loss_ref[...] = jnp.reshape(loss, (1, 1))
    gl_ref[...] = jnp.reshape(global_loss, (1, 1))
    loc_ref[...] = jnp.reshape(loc, (1, 1))
    feat_ref[...] = jnp.reshape(feat, (1, 1))


@functools.partial(jax.jit, static_argnames=("interpret",))
def _run(x1_maps, x2_maps, x1_glob, x2_glob, x1_locations, x2_locations,
         interpret=False):
    fspec = pl.BlockSpec((1, 1, P), lambda b: (b, 0, 0))
    knn = pl.pallas_call(
        _knn_body,
        grid=(B,),
        in_specs=[
            pl.BlockSpec((1, P, D_LOC), lambda b: (b, 0, 0)),
            pl.BlockSpec((1, P, D_LOC), lambda b: (b, 0, 0)),
            pl.BlockSpec((1, P, 2), lambda b: (b, 0, 0)),
            pl.BlockSpec((1, P, 2), lambda b: (b, 0, 0)),
        ],
        out_specs=[fspec] * 8,
        out_shape=(
            [jax.ShapeDtypeStruct((B, 1, P), jnp.float32),
             jax.ShapeDtypeStruct((B, 1, P), jnp.int32)] * 4),
        interpret=interpret,
    )
    (frmin, frarg, fcmin, fcarg,
     lrmin, lrarg, lcmin, lcarg) = knn(
        x1_maps, x2_maps, x1_locations, x2_locations)

    mins = [a.reshape(B, P) for a in
            (frmin, frarg, fcmin, fcarg, lrmin, lrarg, lcmin, lcarg)]
    ch1 = x1_maps[:, :, 0]
    ch2 = x2_maps[:, :, 0]

    out = pl.pallas_call(
        _loss_body,
        out_shape=[jax.ShapeDtypeStruct((1, 1), jnp.float32)] * 4,
        interpret=interpret,
    )(*mins, ch1, ch2, x1_glob, x2_glob)
    loss, gl, loc, feat = (o[0, 0] for o in out)
    return loss, gl, loc, feat


def kernel(x1_maps, x2_maps, x1_glob, x2_glob, x1_locations, x2_locations):
    return _run(x1_maps, x2_maps, x1_glob, x2_glob,
                x1_locations, x2_locations)


# speedup vs baseline: 2.4016x; 1.0769x over previous
import functools

import jax
import jax.numpy as jnp
from jax.experimental import pallas as pl

NUM_MATCHES = 20
ALPHA = 0.75
INV_COEFF = 25.0
VAR_COEFF = 15.0
COV_COEFF = 1.0
GAMMA = 1.0

B, P, D_LOC, D_GLOB = 32, 576, 384, 2048
BIG_I32 = 1 << 30


def _dist_matrix(a, b):
    a2 = jnp.sum(a * a, axis=1, keepdims=True)
    b2 = jnp.transpose(jnp.sum(b * b, axis=1, keepdims=True))
    ab = jax.lax.dot_general(
        a.astype(jnp.bfloat16), b.astype(jnp.bfloat16),
        (((1,), (1,)), ((), ())),
        preferred_element_type=jnp.float32)
    d2 = a2 + b2 - 2.0 * ab
    return jnp.sqrt(jnp.maximum(d2, 0.0))


def _min_argmin(d, axis):
    iota = jax.lax.broadcasted_iota(jnp.int32, d.shape, axis)
    m = jnp.min(d, axis=axis, keepdims=True)
    arg = jnp.min(jnp.where(d == m, iota, BIG_I32), axis=axis)
    return jnp.min(d, axis=axis), arg


def _knn_body(x1m_ref, x2m_ref, x1l_ref, x2l_ref,
              frmin_ref, frarg_ref, fcmin_ref, fcarg_ref,
              lrmin_ref, lrarg_ref, lcmin_ref, lcarg_ref):
    fd = _dist_matrix(x1m_ref[0], x2m_ref[0])
    frmin_ref[0, 0, :], frarg_ref[0, 0, :] = _min_argmin(fd, 1)
    fcmin_ref[0, 0, :], fcarg_ref[0, 0, :] = _min_argmin(fd, 0)
    ld = _dist_matrix(x1l_ref[0], x2l_ref[0])
    lrmin_ref[0, 0, :], lrarg_ref[0, 0, :] = _min_argmin(ld, 1)
    lcmin_ref[0, 0, :], lcarg_ref[0, 0, :] = _min_argmin(ld, 0)


def _match_terms(vals, args, ch_in, ch_cand):
    iota = jax.lax.broadcasted_iota(jnp.int32, (B, P), 1)
    inv_acc = jnp.float32(0.0)
    vin_acc = jnp.float32(0.0)
    vcand_acc = jnp.float32(0.0)
    v = vals
    for _ in range(NUM_MATCHES):
        m = jnp.min(v, axis=1, keepdims=True)
        first = jnp.min(jnp.where(v == m, iota, BIG_I32),
                        axis=1, keepdims=True)
        onehot = iota == first
        fi = jnp.sum(jnp.where(onehot, ch_in, 0.0), axis=1, keepdims=True)
        cand = jnp.sum(jnp.where(onehot, args, 0), axis=1, keepdims=True)
        onehot2 = iota == cand
        fc = jnp.sum(jnp.where(onehot2, ch_cand, 0.0), axis=1, keepdims=True)
        inv_acc += jnp.sum((fi - fc) ** 2)
        mu_i = jnp.sum(fi) / B
        vin_acc += jnp.maximum(
            GAMMA - jnp.sqrt(jnp.sum((fi - mu_i) ** 2) / (B - 1)), 0.0)
        mu_c = jnp.sum(fc) / B
        vcand_acc += jnp.maximum(
            GAMMA - jnp.sqrt(jnp.sum((fc - mu_c) ** 2) / (B - 1)), 0.0)
        v = jnp.where(onehot, jnp.inf, v)
    inv = INV_COEFF * inv_acc / (B * NUM_MATCHES)
    var = VAR_COEFF * (vin_acc + vcand_acc) / (2.0 * NUM_MATCHES)
    return inv + var


def _global_half(x):
    xc = x - jnp.sum(x, axis=0, keepdims=True) / B
    xcc = xc - jnp.sum(xc, axis=0, keepdims=True) / B
    std = jnp.sqrt(jnp.sum(xcc * xcc, axis=0, keepdims=True) / (B - 1))
    var = jnp.sum(jnp.maximum(GAMMA - std, 0.0)) / D_GLOB
    xcb = xc.astype(jnp.bfloat16)
    xcb32 = xcb.astype(jnp.float32)
    colss = jnp.sum(xcb32 * xcb32, axis=0, keepdims=True)
    g = jax.lax.dot_general(
        xcb, xcb, (((1,), (1,)), ((), ())),
        preferred_element_type=jnp.float32)
    s_all = jnp.sum(g * g)
    s_diag = jnp.sum(colss * colss)
    cov = (s_all - s_diag) / ((B - 1.0) * (B - 1.0)) / D_GLOB
    return var, cov


def _loss_body(frmin_ref, frarg_ref, fcmin_ref, fcarg_ref,
               lrmin_ref, lrarg_ref, lcmin_ref, lcarg_ref,
               ch1_ref, ch2_ref, g1_ref, g2_ref,
               loss_ref, gl_ref, loc_ref, feat_ref):
    ch1 = ch1_ref[...]
    ch2 = ch2_ref[...]
    f12 = _match_terms(frmin_ref[...], frarg_ref[...], ch1, ch2)
    f21 = _match_terms(fcmin_ref[...], fcarg_ref[...], ch2, ch1)
    l12 = _match_terms(lrmin_ref[...], lrarg_ref[...], ch1, ch2)
    l21 = _match_terms(lcmin_ref[...], lcarg_ref[...], ch2, ch1)
    feat = (f12 + f21) / 2.0
    loc = (l12 + l21) / 2.0

    g1 = g1_ref[...]
    g2 = g2_ref[...]
    inv_g = INV_COEFF * jnp.sum((g1 - g2) ** 2) / (B * D_GLOB)
    var1, cov1 = _global_half(g1)
    var2, cov2 = _global_half(g2)
    global_loss = (inv_g + VAR_COEFF * (var1 + var2) / 2.0
                   + COV_COEFF * (cov1 + cov2) / 2.0)

    loss = ALPHA * global_loss + (1.0 - ALPHA) * (feat + loc) / 2.0
    loss_ref[...] = jnp.reshape(loss, (1, 1))
    gl_ref[...] = jnp.reshape(global_loss, (1, 1))
    loc_ref[...] = jnp.reshape(loc, (1, 1))
    feat_ref[...] = jnp.reshape(feat, (1, 1))


@functools.partial(jax.jit, static_argnames=("interpret",))
def _run(x1_maps, x2_maps, x1_glob, x2_glob, x1_locations, x2_locations,
         interpret=False):
    fspec = pl.BlockSpec((1, 1, P), lambda b: (b, 0, 0))
    knn = pl.pallas_call(
        _knn_body,
        grid=(B,),
        in_specs=[
            pl.BlockSpec((1, P, D_LOC), lambda b: (b, 0, 0)),
            pl.BlockSpec((1, P, D_LOC), lambda b: (b, 0, 0)),
            pl.BlockSpec((1, P, 2), lambda b: (b, 0, 0)),
            pl.BlockSpec((1, P, 2), lambda b: (b, 0, 0)),
        ],
        out_specs=[fspec] * 8,
        out_shape=(
            [jax.ShapeDtypeStruct((B, 1, P), jnp.float32),
             jax.ShapeDtypeStruct((B, 1, P), jnp.int32)] * 4),
        interpret=interpret,
    )
    (frmin, frarg, fcmin, fcarg,
     lrmin, lrarg, lcmin, lcarg) = knn(
        x1_maps, x2_maps, x1_locations, x2_locations)

    mins = [a.reshape(B, P) for a in
            (frmin, frarg, fcmin, fcarg, lrmin, lrarg, lcmin, lcarg)]
    ch1 = x1_maps[:, :, 0]
    ch2 = x2_maps[:, :, 0]

    out = pl.pallas_call(
        _loss_body,
        out_shape=[jax.ShapeDtypeStruct((1, 1), jnp.float32)] * 4,
        interpret=interpret,
    )(*mins, ch1, ch2, x1_glob, x2_glob)
    loss, gl, loc, feat = (o[0, 0] for o in out)
    return loss, gl, loc, feat


def kernel(x1_maps, x2_maps, x1_glob, x2_glob, x1_locations, x2_locations):
    return _run(x1_maps, x2_maps, x1_glob, x2_glob,
                x1_locations, x2_locations)


# speedup vs baseline: 2.8366x; 1.1812x over previous
import functools

import jax
import jax.numpy as jnp
from jax.experimental import pallas as pl

NUM_MATCHES = 20
ALPHA = 0.75
INV_COEFF = 25.0
VAR_COEFF = 15.0
COV_COEFF = 1.0
GAMMA = 1.0

B, P, D_LOC, D_GLOB = 32, 576, 384, 2048
BIG_I32 = 1 << 30


def _dist_matrix(a, b):
    a2 = jnp.sum(a * a, axis=1, keepdims=True)
    b2 = jnp.transpose(jnp.sum(b * b, axis=1, keepdims=True))
    nab2 = jax.lax.dot_general(
        a.astype(jnp.bfloat16), b.astype(jnp.bfloat16) * jnp.bfloat16(-2.0),
        (((1,), (1,)), ((), ())),
        preferred_element_type=jnp.float32)
    d2 = (a2 + b2) + nab2
    return jnp.sqrt(jnp.maximum(d2, 0.0))


def _min_argmin(d, axis):
    iota = jax.lax.broadcasted_iota(jnp.int32, d.shape, axis)
    m = jnp.min(d, axis=axis, keepdims=True)
    arg = jnp.min(jnp.where(d == m, iota, BIG_I32), axis=axis)
    return jnp.min(d, axis=axis), arg


def _knn_body(x1m_ref, x2m_ref, x1l_ref, x2l_ref,
              frmin_ref, frarg_ref, fcmin_ref, fcarg_ref,
              lrmin_ref, lrarg_ref, lcmin_ref, lcarg_ref):
    fd = _dist_matrix(x1m_ref[0], x2m_ref[0])
    frmin_ref[0, 0, :], frarg_ref[0, 0, :] = _min_argmin(fd, 1)
    fcmin_ref[0, 0, :], fcarg_ref[0, 0, :] = _min_argmin(fd, 0)
    ld = _dist_matrix(x1l_ref[0], x2l_ref[0])
    lrmin_ref[0, 0, :], lrarg_ref[0, 0, :] = _min_argmin(ld, 1)
    lcmin_ref[0, 0, :], lcarg_ref[0, 0, :] = _min_argmin(ld, 0)


def _match_terms4(vals, args, ch_in, ch_cand):
    s = 4 * B
    iota = jax.lax.broadcasted_iota(jnp.int32, (s, P), 1)
    v = vals
    fis = []
    fcs = []
    for _ in range(NUM_MATCHES):
        m = jnp.min(v, axis=1, keepdims=True)
        first = jnp.min(jnp.where(v == m, iota, BIG_I32),
                        axis=1, keepdims=True)
        onehot = iota == first
        fi = jnp.sum(jnp.where(onehot, ch_in, 0.0), axis=1, keepdims=True)
        cand = jnp.sum(jnp.where(onehot, args, 0), axis=1, keepdims=True)
        fc = jnp.sum(jnp.where(iota == cand, ch_cand, 0.0),
                     axis=1, keepdims=True)
        fis.append(fi)
        fcs.append(fc)
        v = jnp.where(onehot, jnp.inf, v)
    fi_all = jnp.concatenate(fis, axis=1)
    fc_all = jnp.concatenate(fcs, axis=1)
    terms = []
    for d in range(4):
        fi = fi_all[d * B:(d + 1) * B]
        fc = fc_all[d * B:(d + 1) * B]
        inv = INV_COEFF * jnp.sum((fi - fc) ** 2) / (B * NUM_MATCHES)
        mu_i = jnp.sum(fi, axis=0, keepdims=True) / B
        std_i = jnp.sqrt(jnp.sum((fi - mu_i) ** 2, axis=0,
                                 keepdims=True) / (B - 1))
        mu_c = jnp.sum(fc, axis=0, keepdims=True) / B
        std_c = jnp.sqrt(jnp.sum((fc - mu_c) ** 2, axis=0,
                                 keepdims=True) / (B - 1))
        var = VAR_COEFF * (jnp.sum(jnp.maximum(GAMMA - std_i, 0.0))
                           + jnp.sum(jnp.maximum(GAMMA - std_c, 0.0))
                           ) / (2.0 * NUM_MATCHES)
        terms.append(inv + var)
    return terms


def _global_half(x):
    xc = x - jnp.sum(x, axis=0, keepdims=True) / B
    xcc = xc - jnp.sum(xc, axis=0, keepdims=True) / B
    std = jnp.sqrt(jnp.sum(xcc * xcc, axis=0, keepdims=True) / (B - 1))
    var = jnp.sum(jnp.maximum(GAMMA - std, 0.0)) / D_GLOB
    xcb = xc.astype(jnp.bfloat16)
    xcb32 = xcb.astype(jnp.float32)
    colss = jnp.sum(xcb32 * xcb32, axis=0, keepdims=True)
    g = jax.lax.dot_general(
        xcb, xcb, (((1,), (1,)), ((), ())),
        preferred_element_type=jnp.float32)
    s_all = jnp.sum(g * g)
    s_diag = jnp.sum(colss * colss)
    cov = (s_all - s_diag) / ((B - 1.0) * (B - 1.0)) / D_GLOB
    return var, cov


def _loss_body(vals_ref, args_ref, chin_ref, chcand_ref, g1_ref, g2_ref,
               loss_ref, gl_ref, loc_ref, feat_ref):
    f12, f21, l12, l21 = _match_terms4(
        vals_ref[...], args_ref[...], chin_ref[...], chcand_ref[...])
    feat = (f12 + f21) / 2.0
    loc = (l12 + l21) / 2.0

    g1 = g1_ref[...]
    g2 = g2_ref[...]
    inv_g = INV_COEFF * jnp.sum((g1 - g2) ** 2) / (B * D_GLOB)
    var1, cov1 = _global_half(g1)
    var2, cov2 = _global_half(g2)
    global_loss = (inv_g + VAR_COEFF * (var1 + var2) / 2.0
                   + COV_COEFF * (cov1 + cov2) / 2.0)

    loss = ALPHA * global_loss + (1.0 - ALPHA) * (feat + loc) / 2.0
    loss_ref[...] = jnp.reshape(loss, (1, 1))
    gl_ref[...] = jnp.reshape(global_loss, (1, 1))
    loc_ref[...] = jnp.reshape(loc, (1, 1))
    feat_ref[...] = jnp.reshape(feat, (1, 1))


@functools.partial(jax.jit, static_argnames=("interpret",))
def _run(x1_maps, x2_maps, x1_glob, x2_glob, x1_locations, x2_locations,
         interpret=False):
    fspec = pl.BlockSpec((1, 1, P), lambda b: (b, 0, 0))
    knn = pl.pallas_call(
        _knn_body,
        grid=(B,),
        in_specs=[
            pl.BlockSpec((1, P, D_LOC), lambda b: (b, 0, 0)),
            pl.BlockSpec((1, P, D_LOC), lambda b: (b, 0, 0)),
            pl.BlockSpec((1, P, 2), lambda b: (b, 0, 0)),
            pl.BlockSpec((1, P, 2), lambda b: (b, 0, 0)),
        ],
        out_specs=[fspec] * 8,
        out_shape=(
            [jax.ShapeDtypeStruct((B, 1, P), jnp.float32),
             jax.ShapeDtypeStruct((B, 1, P), jnp.int32)] * 4),
        interpret=interpret,
    )
    (frmin, frarg, fcmin, fcarg,
     lrmin, lrarg, lcmin, lcarg) = knn(
        x1_maps, x2_maps, x1_locations, x2_locations)

    vals = jnp.concatenate(
        [a.reshape(B, P) for a in (frmin, fcmin, lrmin, lcmin)], axis=0)
    args = jnp.concatenate(
        [a.reshape(B, P) for a in (frarg, fcarg, lrarg, lcarg)], axis=0)
    ch1 = x1_maps[:, :, 0]
    ch2 = x2_maps[:, :, 0]
    chin = jnp.concatenate([ch1, ch2, ch1, ch2], axis=0)
    chcand = jnp.concatenate([ch2, ch1, ch2, ch1], axis=0)

    out = pl.pallas_call(
        _loss_body,
        out_shape=[jax.ShapeDtypeStruct((1, 1), jnp.float32)] * 4,
        interpret=interpret,
    )(vals, args, chin, chcand, x1_glob, x2_glob)
    loss, gl, loc, feat = (o[0, 0] for o in out)
    return loss, gl, loc, feat


def kernel(x1_maps, x2_maps, x1_glob, x2_glob, x1_locations, x2_locations):
    return _run(x1_maps, x2_maps, x1_glob, x2_glob,
                x1_locations, x2_locations)
